# Initial kernel scaffold; baseline (speedup 1.0000x reference)
import jax
import jax.numpy as jnp
from jax.experimental import pallas as pl


def kernel(num_queries, num_keys, weight):
    raise NotImplementedError("write your pallas kernel here")


# speedup vs baseline: 53.0144x; 53.0144x over previous
import functools
import math

import jax
import jax.numpy as jnp
from jax import lax
from jax.experimental import pallas as pl
from jax.experimental.pallas import tpu as pltpu
from jax.experimental.pallas import tpu_sc as plsc

H = 12
Q = 2048
K = 2048
NB = 32
TW = 4112
NSHIFT = 8
HP = 16
_LOG_RATIO = math.log(128 / 8)

NW = 32
ROWS = H * Q
RPW = ROWS // NW


def _table_kernel(s_ref, wt_ref, out_ref):
    t = pl.program_id(0)
    d = lax.broadcasted_iota(jnp.int32, (1, TW), 1) + (t - (Q - 1) + s_ref[0])
    ret = (d >= 0).astype(jnp.int32) * (NB // 2)
    n = jnp.abs(d)
    max_exact = NB // 4
    n_safe = jnp.maximum(n, 1)
    val_if_large = max_exact + (
        jnp.log(n_safe.astype(jnp.float32) / max_exact)
        / _LOG_RATIO
        * (NB // 2 - max_exact)
    ).astype(jnp.int32)
    val_if_large = jnp.minimum(val_if_large, NB // 2 - 1)
    bucket = ret + jnp.where(n < max_exact, n, val_if_large)
    b_iota = lax.broadcasted_iota(jnp.int32, (NB, TW), 0)
    onehot = (bucket == b_iota).astype(jnp.float32)
    out_ref[...] = jnp.dot(wt_ref[...], onehot, preferred_element_type=jnp.float32)


def _build_table(s, weight_t):
    return pl.pallas_call(
        _table_kernel,
        grid=(NSHIFT,),
        in_specs=[
            pl.BlockSpec(memory_space=pltpu.SMEM),
            pl.BlockSpec((HP, NB), lambda t: (0, 0)),
        ],
        out_specs=pl.BlockSpec((HP, TW), lambda t: (t, 0)),
        out_shape=jax.ShapeDtypeStruct((NSHIFT * HP, TW), jnp.float32),
    )(s, weight_t)


def _expand_body(table_hbm, out_hbm, v8, sem):
    wid = lax.axis_index("s") * 2 + lax.axis_index("c")
    r0 = wid * RPW
    h0 = r0 // Q
    h1 = (r0 + RPW - 1) // Q
    for t in range(NSHIFT):
        pltpu.sync_copy(
            table_hbm.at[pl.ds((t * HP + h0) * TW, TW)],
            v8.at[pl.ds(t * TW, TW)],
        )
        pltpu.sync_copy(
            table_hbm.at[pl.ds((t * HP + h1) * TW, TW)],
            v8.at[pl.ds((NSHIFT + t) * TW, TW)],
        )

    def fire(r, carry):
        h = r // Q
        i = r - h * Q
        start = (Q - 1) - i
        b = lax.rem(start, NSHIFT)
        a8 = start - b
        sel = h - h0
        src_off = pl.multiple_of((sel * NSHIFT + b) * TW + a8, NSHIFT)
        pltpu.make_async_copy(
            v8.at[pl.ds(src_off, K)],
            out_hbm.at[pl.ds(r * K, K)],
            sem,
        ).start()
        return carry

    lax.fori_loop(r0, r0 + RPW, fire, 0)

    def drain(r, carry):
        pltpu.make_async_copy(
            v8.at[pl.ds(0, K)],
            out_hbm.at[pl.ds(r0 * K, K)],
            sem,
        ).wait()
        return carry

    lax.fori_loop(0, RPW, drain, 0)


@functools.partial(jax.jit, static_argnames=())
def _expand(table):
    kern = pl.kernel(
        _expand_body,
        out_type=jax.ShapeDtypeStruct((ROWS * K,), jnp.float32),
        mesh=plsc.VectorSubcoreMesh(core_axis_name="c", subcore_axis_name="s"),
        scratch_types=[
            pltpu.VMEM((2 * NSHIFT * TW,), jnp.float32),
            pltpu.SemaphoreType.DMA,
        ],
    )
    return kern(table.reshape(NSHIFT * HP * TW))


def kernel(num_queries, num_keys, weight):
    s = (jnp.asarray(num_queries, jnp.int32) - jnp.int32(Q)).reshape(1)
    weight_t = jnp.zeros((HP, NB), jnp.float32).at[:H, :].set(weight.T)
    table = _build_table(s, weight_t)
    out = _expand(table)
    return out.reshape(1, H, Q, K)
